# baseline (speedup 1.0000x reference)
import jax
import jax.numpy as jnp
from jax import lax
from jax.experimental import pallas as pl
from jax.experimental.pallas import tpu as pltpu
from jax.experimental.pallas import tpu_sc as plsc

_NC = 2
_NS = 16
_NW = _NC * _NS
_B, _S, _D = 1024, 200, 128
_N = _B * _S
_PER_W = _N // _NW
_C = 40
_NCH = _PER_W // _C
_DV = _D // 16
_R = 8
_SS = 4
_P = 4


def _body(seq_hbm, table_hbm, type_hbm, pos_hbm, out_hbm,
          idx_all, bufs, type_v, bias_sh, stage_sh, *sems):
    gsems = sems[:_R]
    asems = sems[_R:_R + _SS]
    bsems = sems[_R + _SS:]
    sid = lax.axis_index("s")
    wid = sid * _NC + lax.axis_index("c")
    base = wid * _PER_W

    pltpu.sync_copy(seq_hbm.at[pl.ds(wid * _NCH, _NCH)], idx_all)

    @pl.when(sid == 0)
    def _mk_bias():
        pltpu.sync_copy(type_hbm, type_v)
        for off, n in ((0, 80), (80, 80), (160, 40)):
            pltpu.sync_copy(pos_hbm.at[pl.ds(off, n)],
                            bufs.at[0, pl.ds(0, n)])

            @pl.loop(0, n)
            def _add_type(r):
                for t in range(_DV):
                    sl = pl.ds(t * 16, 16)
                    bufs[0, r, sl] = bufs[0, r, sl] + type_v[0, sl]

            pltpu.sync_copy(bufs.at[0, pl.ds(0, n)],
                            bias_sh.at[pl.ds(off, n)])
            pltpu.sync_copy(bufs.at[0, pl.ds(0, n)],
                            bias_sh.at[pl.ds(off + _S, n)])

    plsc.subcore_barrier()

    def prefill(c, b):
        off = lax.rem(c * _C, _S)
        pltpu.sync_copy(bias_sh.at[pl.ds(off, _C)], bufs.at[b])

    def start_gather(c, b):
        pltpu.async_copy(table_hbm.at[idx_all.at[c]], bufs.at[b],
                         gsems[b], add=True)

    def wait_gather(b):
        pltpu.make_async_copy(table_hbm.at[idx_all.at[0]], bufs.at[b],
                              gsems[b]).wait()

    def slot(sb):
        return stage_sh.at[sid * _SS + sb]

    def start_stage(b, sb):
        pltpu.async_copy(bufs.at[b], slot(sb), asems[sb])

    def wait_stage(sb):
        pltpu.make_async_copy(bufs.at[0], slot(sb), asems[sb]).wait()

    def start_write(c, sb):
        pltpu.async_copy(slot(sb), out_hbm.at[pl.ds(base + c * _C, _C)],
                         bsems[sb])

    def wait_write(sb):
        pltpu.make_async_copy(slot(sb), out_hbm.at[pl.ds(0, _C)],
                              bsems[sb]).wait()

    for b in range(_P):
        prefill(b, b)
        start_gather(b, b)

    @pl.loop(0, _NCH // _R)
    def _outer(i):
        for b in range(_R):
            k = i * _R + b
            sb = b % _SS
            sbprev = (b + _SS - 1) % _SS
            wait_gather(b)

            @pl.when(k >= _SS)
            def _wslot():
                wait_write(sb)

            start_stage(b, sb)

            @pl.when(k >= 1)
            def _push():
                wait_stage(sbprev)
                start_write(k - 1, sbprev)

            c = k + _P
            b2 = (b + _P) % _R

            @pl.when(c < _NCH)
            def _prep():
                prefill(c, b2)
                start_gather(c, b2)

    wait_stage((_NCH - 1) % _SS)
    start_write(_NCH - 1, (_NCH - 1) % _SS)
    for sb in range(_SS):
        wait_write(sb)


@jax.jit
def kernel(sequence, token_table, type_table, pos_table):
    seq2 = sequence.reshape(_N // _C, _C)
    mesh = plsc.VectorSubcoreMesh(core_axis_name="c", subcore_axis_name="s")
    out = pl.kernel(
        _body,
        out_type=jax.ShapeDtypeStruct((_N, _D), jnp.float32),
        mesh=mesh,
        scratch_types=[
            pltpu.VMEM((_NCH, _C), jnp.int32),
            pltpu.VMEM((_R, _C, _D), jnp.float32),
            pltpu.VMEM((1, _D), jnp.float32),
            pltpu.VMEM_SHARED((2 * _S, _D), jnp.float32),
            pltpu.VMEM_SHARED((_NS * _SS, _C, _D), jnp.float32),
        ] + [pltpu.SemaphoreType.DMA] * (_R + 2 * _SS),
    )(seq2, token_table, type_table, pos_table)
    return out.reshape(_B, _S, _D)


# speedup vs baseline: 1.2498x; 1.2498x over previous
import jax
import jax.numpy as jnp
from jax import lax
from jax.experimental import pallas as pl
from jax.experimental.pallas import tpu as pltpu
from jax.experimental.pallas import tpu_sc as plsc

_NC = 2
_NS = 16
_NW = _NC * _NS
_B, _S, _D = 1024, 200, 128
_N = _B * _S
_PER_W = _N // _NW
_C = _S
_NCH = _PER_W // _C
_DV = _D // 16
_R = 4


def _body(seq_hbm, table_hbm, type_hbm, pos_hbm, out_hbm,
          idx_all, bufs, type_v, bias_sh,
          gs0, gs1, gs2, gs3, ss0, ss1, ss2, ss3):
    gsems = (gs0, gs1, gs2, gs3)
    ssems = (ss0, ss1, ss2, ss3)
    sid = lax.axis_index("s")
    wid = sid * _NC + lax.axis_index("c")
    base = wid * _PER_W

    pltpu.sync_copy(seq_hbm.at[pl.ds(wid * _NCH, _NCH)], idx_all)

    @pl.when(sid == 0)
    def _mk_bias():
        pltpu.sync_copy(pos_hbm.at[pl.ds(0, _S)], bufs.at[0])
        pltpu.sync_copy(type_hbm, type_v)

        @pl.loop(0, _S)
        def _add_type(r):
            for t in range(_DV):
                sl = pl.ds(t * 16, 16)
                bufs[0, r, sl] = bufs[0, r, sl] + type_v[0, sl]

        pltpu.sync_copy(bufs.at[0], bias_sh)

    plsc.subcore_barrier()

    def prefill(b):
        pltpu.sync_copy(bias_sh, bufs.at[b])

    def start_gather(c, b):
        pltpu.async_copy(table_hbm.at[idx_all.at[c, pl.ds(0, 128)]],
                         bufs.at[b, pl.ds(0, 128)], gsems[b], add=True)
        pltpu.async_copy(table_hbm.at[idx_all.at[c, pl.ds(128, 72)]],
                         bufs.at[b, pl.ds(128, 72)], gsems[b], add=True)

    def wait_gather(b):
        pltpu.make_async_copy(table_hbm.at[idx_all.at[0, pl.ds(0, 128)]],
                              bufs.at[b, pl.ds(0, 128)], gsems[b]).wait()
        pltpu.make_async_copy(table_hbm.at[idx_all.at[0, pl.ds(128, 72)]],
                              bufs.at[b, pl.ds(128, 72)], gsems[b]).wait()

    def start_store(c, b):
        pltpu.async_copy(bufs.at[b], out_hbm.at[pl.ds(base + c * _C, _C)],
                         ssems[b])

    def wait_store(b):
        pltpu.make_async_copy(bufs.at[b], out_hbm.at[pl.ds(0, _C)],
                              ssems[b]).wait()

    for b in range(2):
        prefill(b)
        start_gather(b, b)

    @pl.loop(0, _NCH // _R)
    def _outer(i):
        for b in range(_R):
            k = i * _R + b
            c = k + 2
            b2 = (b + 2) % _R

            @pl.when(c < _NCH)
            def _prep():
                @pl.when(c >= _R)
                def _w():
                    wait_store(b2)
                prefill(b2)
                start_gather(c, b2)

            wait_gather(b)
            start_store(k, b)

    for b in range(_R):
        wait_store(b)


@jax.jit
def kernel(sequence, token_table, type_table, pos_table):
    seq2 = sequence.reshape(_N // _C, _C)
    mesh = plsc.VectorSubcoreMesh(core_axis_name="c", subcore_axis_name="s")
    out = pl.kernel(
        _body,
        out_type=jax.ShapeDtypeStruct((_N, _D), jnp.float32),
        mesh=mesh,
        scratch_types=[
            pltpu.VMEM((_NCH, _C), jnp.int32),
            pltpu.VMEM((_R, _C, _D), jnp.float32),
            pltpu.VMEM((1, _D), jnp.float32),
            pltpu.VMEM_SHARED((_C, _D), jnp.float32),
        ] + [pltpu.SemaphoreType.DMA] * 8,
    )(seq2, token_table, type_table, pos_table)
    return out.reshape(_B, _S, _D)


# speedup vs baseline: 1.2559x; 1.0049x over previous
import jax
import jax.numpy as jnp
from jax import lax
from jax.experimental import pallas as pl
from jax.experimental.pallas import tpu as pltpu
from jax.experimental.pallas import tpu_sc as plsc

_NC = 2
_NS = 16
_NW = _NC * _NS
_B, _S, _D = 1024, 200, 128
_N = _B * _S
_PER_W = _N // _NW
_C = _S
_NCH = _PER_W // _C
_DV = _D // 16
_R = 4


def _body(seq_hbm, table_hbm, type_hbm, pos_hbm, out_hbm,
          idx_all, bufs, type_v, bias_sh,
          gs0, gs1, gs2, gs3, ss0, ss1, ss2, ss3):
    gsems = (gs0, gs1, gs2, gs3)
    ssems = (ss0, ss1, ss2, ss3)
    sid = lax.axis_index("s")
    wid = sid * _NC + lax.axis_index("c")
    base = wid * _PER_W

    pltpu.sync_copy(seq_hbm.at[pl.ds(wid * _NCH, _NCH)], idx_all)

    @pl.when(sid == 0)
    def _mk_bias():
        pltpu.sync_copy(pos_hbm.at[pl.ds(0, _S)], bufs.at[0])
        pltpu.sync_copy(type_hbm, type_v)

        @pl.loop(0, _S)
        def _add_type(r):
            for t in range(_DV):
                sl = pl.ds(t * 16, 16)
                bufs[0, r, sl] = bufs[0, r, sl] + type_v[0, sl]

        pltpu.sync_copy(bufs.at[0], bias_sh)

    plsc.subcore_barrier()

    def prefill(b):
        pltpu.sync_copy(bias_sh, bufs.at[b])

    def start_gather(c, b):
        pltpu.async_copy(table_hbm.at[idx_all.at[c, pl.ds(0, 128)]],
                         bufs.at[b, pl.ds(0, 128)], gsems[b], add=True)
        pltpu.async_copy(table_hbm.at[idx_all.at[c, pl.ds(128, 72)]],
                         bufs.at[b, pl.ds(128, 72)], gsems[b], add=True)

    def wait_gather(b):
        pltpu.make_async_copy(table_hbm.at[idx_all.at[0, pl.ds(0, 128)]],
                              bufs.at[b, pl.ds(0, 128)], gsems[b]).wait()
        pltpu.make_async_copy(table_hbm.at[idx_all.at[0, pl.ds(128, 72)]],
                              bufs.at[b, pl.ds(128, 72)], gsems[b]).wait()

    def start_store(c, b):
        pltpu.async_copy(bufs.at[b], out_hbm.at[pl.ds(base + c * _C, _C)],
                         ssems[b])

    def wait_store(b):
        pltpu.make_async_copy(bufs.at[b], out_hbm.at[pl.ds(0, _C)],
                              ssems[b]).wait()

    for b in range(2):
        prefill(b)
        start_gather(b, b)

    @pl.loop(0, _NCH // _R)
    def _outer(i):
        for b in range(_R):
            k = i * _R + b
            wait_gather(b)
            start_store(k, b)
            c = k + 2
            b2 = (b + 2) % _R

            @pl.when(c < _NCH)
            def _prep():
                @pl.when(c >= _R)
                def _w():
                    wait_store(b2)
                prefill(b2)
                start_gather(c, b2)

    for b in range(_R):
        wait_store(b)


@jax.jit
def kernel(sequence, token_table, type_table, pos_table):
    seq2 = sequence.reshape(_N // _C, _C)
    mesh = plsc.VectorSubcoreMesh(core_axis_name="c", subcore_axis_name="s")
    out = pl.kernel(
        _body,
        out_type=jax.ShapeDtypeStruct((_N, _D), jnp.float32),
        mesh=mesh,
        scratch_types=[
            pltpu.VMEM((_NCH, _C), jnp.int32),
            pltpu.VMEM((_R, _C, _D), jnp.float32),
            pltpu.VMEM((1, _D), jnp.float32),
            pltpu.VMEM_SHARED((_C, _D), jnp.float32),
        ] + [pltpu.SemaphoreType.DMA] * 8,
    )(seq2, token_table, type_table, pos_table)
    return out.reshape(_B, _S, _D)
